# baseline (device time: 74094 ns/iter reference)
import jax
import jax.numpy as jnp
from jax import lax
from jax.experimental import pallas as pl
from jax.experimental.pallas import tpu as pltpu

N_DEV = 32


def kernel(x, Win0, Wout0, Win1, Wout1, Win2, Wout2):
    b, d_per = x.shape
    _, h_dim = Win0.shape
    chunk = b // N_DEV

    def body(x_ref, win0_ref, wout0_ref, win1_ref, wout1_ref, win2_ref,
             wout2_ref, out_ref, pbuf, recv1, recv2, ss1, rs1, ss2, rs2):
        me = lax.axis_index("i")

        def all_reduce_relu(partial_f32):
            pbuf[...] = partial_f32.astype(jnp.bfloat16)
            sends = []
            for off in range(1, N_DEV):
                tgt = lax.rem(me + off, N_DEV)
                d = pltpu.make_async_remote_copy(
                    src_ref=pbuf.at[pl.ds(tgt * chunk, chunk), :],
                    dst_ref=recv1.at[me],
                    send_sem=ss1.at[off - 1],
                    recv_sem=rs1.at[me],
                    device_id=(tgt,),
                    device_id_type=pl.DeviceIdType.MESH,
                )
                d.start()
                sends.append(d)
            recv1[pl.ds(me, 1)] = pbuf[pl.ds(me * chunk, chunk), :][None]
            for off in range(1, N_DEV):
                src = lax.rem(me - off + N_DEV, N_DEV)
                pltpu.make_async_remote_copy(
                    src_ref=pbuf.at[pl.ds(0, chunk), :],
                    dst_ref=recv1.at[src],
                    send_sem=ss1.at[0],
                    recv_sem=rs1.at[src],
                    device_id=(src,),
                    device_id_type=pl.DeviceIdType.MESH,
                ).wait_recv()
            for d in sends:
                d.wait_send()
            s = jnp.sum(recv1[...].astype(jnp.float32), axis=0)
            s = jnp.maximum(s, 0.0).astype(jnp.bfloat16)
            recv2[pl.ds(me, 1)] = s[None]
            sends = []
            for off in range(1, N_DEV):
                tgt = lax.rem(me + off, N_DEV)
                d = pltpu.make_async_remote_copy(
                    src_ref=recv2.at[me],
                    dst_ref=recv2.at[me],
                    send_sem=ss2.at[off - 1],
                    recv_sem=rs2.at[me],
                    device_id=(tgt,),
                    device_id_type=pl.DeviceIdType.MESH,
                )
                d.start()
                sends.append(d)
            for off in range(1, N_DEV):
                src = lax.rem(me - off + N_DEV, N_DEV)
                pltpu.make_async_remote_copy(
                    src_ref=recv2.at[src],
                    dst_ref=recv2.at[src],
                    send_sem=ss2.at[0],
                    recv_sem=rs2.at[src],
                    device_id=(src,),
                    device_id_type=pl.DeviceIdType.MESH,
                ).wait_recv()
            for d in sends:
                d.wait_send()
            return recv2[...].reshape(b, h_dim)

        xb = x_ref[...].astype(jnp.bfloat16)
        xf = None
        for win_ref, wout_ref in ((win0_ref, wout0_ref),
                                  (win1_ref, wout1_ref),
                                  (win2_ref, wout2_ref)):
            wb = win_ref[...].astype(jnp.bfloat16)
            partial = jnp.dot(xb, wb, preferred_element_type=jnp.float32)
            hmat = all_reduce_relu(partial)
            wob = wout_ref[...].astype(jnp.bfloat16)
            xf = jnp.dot(hmat, wob, preferred_element_type=jnp.float32)
            xb = xf.astype(jnp.bfloat16)
        out_ref[...] = xf

    return pl.pallas_call(
        body,
        out_shape=jax.ShapeDtypeStruct((b, d_per), jnp.float32),
        in_specs=[pl.BlockSpec(memory_space=pltpu.VMEM)] * 7,
        out_specs=pl.BlockSpec(memory_space=pltpu.VMEM),
        scratch_shapes=[
            pltpu.VMEM((b, h_dim), jnp.bfloat16),
            pltpu.VMEM((N_DEV, chunk, h_dim), jnp.bfloat16),
            pltpu.VMEM((N_DEV, chunk, h_dim), jnp.bfloat16),
            pltpu.SemaphoreType.DMA((N_DEV - 1,)),
            pltpu.SemaphoreType.DMA((N_DEV,)),
            pltpu.SemaphoreType.DMA((N_DEV - 1,)),
            pltpu.SemaphoreType.DMA((N_DEV,)),
        ],
    )(x, Win0, Wout0, Win1, Wout1, Win2, Wout2)
